# Initial kernel scaffold; baseline (speedup 1.0000x reference)
import jax
import jax.numpy as jnp
from jax.experimental import pallas as pl


def kernel(token_ids, embedding_weight, positional_weight):
    raise NotImplementedError("write your pallas kernel here")


# speedup vs baseline: 1.1079x; 1.1079x over previous
import functools

import jax
import jax.numpy as jnp
from jax import lax
from jax.experimental import pallas as pl
from jax.experimental.pallas import tpu as pltpu
from jax.experimental.pallas import tpu_sc as plsc

_B, _S, _D = 4, 4096, 768
_N = _B * _S
_NC, _NS = 2, 16
_NW = _NC * _NS
_PS = _S // _NW
_C = 64
_NSC = _PS // _C
_NV = _D // 16

_mesh = plsc.VectorSubcoreMesh(core_axis_name="c", subcore_axis_name="s")


@functools.partial(
    pl.kernel,
    mesh=_mesh,
    out_type=jax.ShapeDtypeStruct((_N, _D), jnp.float32),
    scratch_types=[
        pltpu.VMEM((_C,), jnp.int32),
        pltpu.VMEM((_C, _D), jnp.float32),
        pltpu.VMEM((_C, _D), jnp.float32),
        pltpu.SemaphoreType.DMA,
    ],
)
def _emb_lookup(tok_hbm, emb_hbm, pos_hbm, out_hbm, idx_v, pos_v, tok_v, sem):
    wid = lax.axis_index("s") * _NC + lax.axis_index("c")
    s_w = wid * _PS
    for sc in range(_NSC):
        s0 = s_w + sc * _C
        pltpu.sync_copy(pos_hbm.at[pl.ds(s0, _C)], pos_v)
        for b in range(_B):
            r0 = b * _S + s0
            pltpu.sync_copy(tok_hbm.at[pl.ds(r0, _C)], idx_v)
            pltpu.async_copy(emb_hbm.at[idx_v], tok_v, sem).wait()

            def row_add(i, _):
                for j in range(_NV):
                    sl = pl.ds(j * 16, 16)
                    tok_v[i, sl] = tok_v[i, sl] + pos_v[i, sl]
                return 0

            lax.fori_loop(0, _C, row_add, 0)
            pltpu.sync_copy(tok_v, out_hbm.at[pl.ds(r0, _C)])


def kernel(token_ids, embedding_weight, positional_weight):
    tok = jnp.reshape(token_ids.astype(jnp.int32), (_N,))
    out = _emb_lookup(tok, embedding_weight, positional_weight)
    return jnp.reshape(out, (_B, _S, _D))
